# Initial kernel scaffold; baseline (speedup 1.0000x reference)
import functools

import jax
import jax.numpy as jnp
from jax import lax
from jax.experimental import pallas as pl
from jax.experimental.pallas import tpu as pltpu
from jax.experimental.pallas import tpu_sc as plsc

_D = 2
_DD = _D * _D
_NC = 2
_NS = 16
_NW = _NC * _NS


def _proj_body(m_ref, w_ref, b_ref, o_ref):
    acc = jax.lax.dot_general(
        m_ref[...], w_ref[...], (((1,), (1,)), ((), ())),
        preferred_element_type=jnp.float32)
    o_ref[...] = 0.5 * acc + b_ref[...]


def _project(m, w, b2, bm):
    rows = m.shape[0]
    return pl.pallas_call(
        _proj_body,
        grid=(rows // bm,),
        in_specs=[
            pl.BlockSpec((bm, m.shape[1]), lambda i: (i, 0)),
            pl.BlockSpec(w.shape, lambda i: (0, 0)),
            pl.BlockSpec(b2.shape, lambda i: (0, 0)),
        ],
        out_specs=pl.BlockSpec((bm, _DD), lambda i: (i, 0)),
        out_shape=jax.ShapeDtypeStruct((rows, _DD), jnp.float32),
    )(m, w, b2)


def _sc_build(nnz, n_px, n_pe):
    per_w = nnz // _NW
    chunk = 2000
    while per_w % chunk:
        chunk //= 2
    groups = chunk // 16

    mesh = plsc.VectorSubcoreMesh(
        core_axis_name="c", subcore_axis_name="s",
        num_cores=_NC, num_subcores=_NS)

    @functools.partial(
        pl.kernel,
        out_type=[
            jax.ShapeDtypeStruct((2 * _DD * nnz,), jnp.int32),
            jax.ShapeDtypeStruct((_DD * nnz,), jnp.float32),
        ],
        mesh=mesh,
        scratch_types=[
            pltpu.VMEM((n_px,), jnp.float32),
            pltpu.VMEM((n_pe,), jnp.float32),
            pltpu.VMEM((chunk,), jnp.int32),
            pltpu.VMEM((chunk,), jnp.int32),
            pltpu.VMEM((_DD * chunk,), jnp.int32),
            pltpu.VMEM((_DD * chunk,), jnp.int32),
            pltpu.VMEM((_DD * chunk,), jnp.float32),
        ],
    )
    def sc_fn(px_hbm, pe_hbm, row_hbm, col_hbm, idx_hbm, attr_hbm,
              px_v, pe_v, row_v, col_v, i0_v, i1_v, at_v):
        wid = lax.axis_index("s") * _NC + lax.axis_index("c")
        base = wid * per_w
        pltpu.sync_copy(px_hbm, px_v)
        pltpu.sync_copy(pe_hbm, pe_v)
        lane4 = lax.iota(jnp.int32, 16) * _DD

        for c in range(per_w // chunk):
            off = base + c * chunk
            pltpu.sync_copy(row_hbm.at[pl.ds(off, chunk)], row_v)
            pltpu.sync_copy(col_hbm.at[pl.ds(off, chunk)], col_v)

            def body(k, carry):
                s = pl.ds(k * 16, 16)
                rv = row_v[s]
                cv = col_v[s]
                r4 = rv * _DD
                c4 = cv * _DD
                r2 = rv * _D
                c2 = cv * _D
                pos0 = k * (16 * _DD) + lane4
                for j in range(_DD):
                    pxj = plsc.load_gather(px_v, [r4 + j])
                    pej = plsc.load_gather(pe_v, [c4 + j])
                    sgd = 1.0 / (1.0 + jnp.exp(-(pxj + pej)))
                    pos = pos0 + j
                    plsc.store_scatter(at_v, [pos], sgd)
                    plsc.store_scatter(i0_v, [pos], r2 + (j // _D))
                    plsc.store_scatter(i1_v, [pos], c2 + (j % _D))
                return carry

            lax.fori_loop(0, groups, body, 0)

            obase = off * _DD
            pltpu.sync_copy(at_v, attr_hbm.at[pl.ds(obase, chunk * _DD)])
            pltpu.sync_copy(i0_v, idx_hbm.at[pl.ds(obase, chunk * _DD)])
            pltpu.sync_copy(
                i1_v, idx_hbm.at[pl.ds(_DD * nnz + obase, chunk * _DD)])

    return sc_fn


def kernel(x, e, hyperedge_index, W, b):
    f = x.shape[-1]
    x2 = x.reshape(x.shape[0] // _D, _D * f)
    e2 = e.reshape(e.shape[0] // _D, _D * f)
    wx = jnp.concatenate([W[:, :f]] * _D, axis=1)
    we = jnp.concatenate([W[:, f:]] * _D, axis=1)
    zb = jnp.zeros((1, _DD), jnp.float32)
    b2 = b.reshape(1, _DD).astype(jnp.float32)

    px = _project(x2, wx, zb, 1000)
    pe = _project(e2, we, b2, 1000)

    row = hyperedge_index[0]
    col = hyperedge_index[1]
    nnz = row.shape[0]

    sc_fn = _sc_build(nnz, px.size, pe.size)
    idx_flat, attr = sc_fn(px.reshape(-1), pe.reshape(-1), row, col)
    return idx_flat.reshape(2, _DD * nnz), attr


# speedup vs baseline: 13.5947x; 13.5947x over previous
import functools

import jax
import jax.numpy as jnp
from jax import lax
from jax.experimental import pallas as pl
from jax.experimental.pallas import tpu as pltpu
from jax.experimental.pallas import tpu_sc as plsc

_D = 2
_DD = _D * _D
_NC = 2
_NS = 16
_NW = _NC * _NS


def _proj_body(m_ref, w_ref, b_ref, o_ref):
    acc = jax.lax.dot_general(
        m_ref[...], w_ref[...], (((1,), (1,)), ((), ())),
        preferred_element_type=jnp.float32)
    o_ref[...] = 0.5 * acc + b_ref[...]


def _project(m, w, b2, bm):
    rows = m.shape[0]
    return pl.pallas_call(
        _proj_body,
        grid=(rows // bm,),
        in_specs=[
            pl.BlockSpec((bm, m.shape[1]), lambda i: (i, 0)),
            pl.BlockSpec(w.shape, lambda i: (0, 0)),
            pl.BlockSpec(b2.shape, lambda i: (0, 0)),
        ],
        out_specs=pl.BlockSpec((bm, _DD), lambda i: (i, 0)),
        out_shape=jax.ShapeDtypeStruct((rows, _DD), jnp.float32),
    )(m, w, b2)


def _sc_build(nnz, n_px, n_pe):
    per_w = nnz // _NW
    chunk = 2000
    while per_w % chunk:
        chunk //= 2
    groups = chunk // 16

    mesh = plsc.VectorSubcoreMesh(
        core_axis_name="c", subcore_axis_name="s",
        num_cores=_NC, num_subcores=_NS)

    @functools.partial(
        pl.kernel,
        out_type=[
            jax.ShapeDtypeStruct((2 * _DD * nnz,), jnp.int32),
            jax.ShapeDtypeStruct((_DD * nnz,), jnp.float32),
        ],
        mesh=mesh,
        compiler_params=pltpu.CompilerParams(needs_layout_passes=False),
        scratch_types=[
            pltpu.VMEM((n_px,), jnp.float32),
            pltpu.VMEM((n_pe,), jnp.float32),
            pltpu.VMEM((chunk,), jnp.int32),
            pltpu.VMEM((chunk,), jnp.int32),
            pltpu.VMEM((_DD * chunk,), jnp.int32),
            pltpu.VMEM((_DD * chunk,), jnp.int32),
            pltpu.VMEM((_DD * chunk,), jnp.float32),
        ],
    )
    def sc_fn(px_hbm, pe_hbm, row_hbm, col_hbm, idx_hbm, attr_hbm,
              px_v, pe_v, row_v, col_v, i0_v, i1_v, at_v):
        wid = lax.axis_index("s") * _NC + lax.axis_index("c")
        base = wid * per_w
        pltpu.sync_copy(px_hbm, px_v)
        pltpu.sync_copy(pe_hbm, pe_v)
        lane4 = lax.iota(jnp.int32, 16) * _DD

        for c in range(per_w // chunk):
            off = base + c * chunk
            pltpu.sync_copy(row_hbm.at[pl.ds(off, chunk)], row_v)
            pltpu.sync_copy(col_hbm.at[pl.ds(off, chunk)], col_v)

            def body(k, carry):
                s = pl.ds(k * 16, 16)
                rv = row_v[s]
                cv = col_v[s]
                r4 = rv * _DD
                c4 = cv * _DD
                r2 = rv * _D
                c2 = cv * _D
                pos0 = k * (16 * _DD) + lane4
                for j in range(_DD):
                    pxj = plsc.load_gather(px_v, [r4 + j])
                    pej = plsc.load_gather(pe_v, [c4 + j])
                    sgd = 1.0 / (1.0 + jnp.exp(-(pxj + pej)))
                    pos = pos0 + j
                    plsc.store_scatter(at_v, [pos], sgd)
                    plsc.store_scatter(i0_v, [pos], r2 + (j // _D))
                    plsc.store_scatter(i1_v, [pos], c2 + (j % _D))
                return carry

            lax.fori_loop(0, groups, body, 0)

            obase = off * _DD
            pltpu.sync_copy(at_v, attr_hbm.at[pl.ds(obase, chunk * _DD)])
            pltpu.sync_copy(i0_v, idx_hbm.at[pl.ds(obase, chunk * _DD)])
            pltpu.sync_copy(
                i1_v, idx_hbm.at[pl.ds(_DD * nnz + obase, chunk * _DD)])

    return sc_fn


def kernel(x, e, hyperedge_index, W, b):
    f = x.shape[-1]
    x2 = x.reshape(x.shape[0] // _D, _D * f)
    e2 = e.reshape(e.shape[0] // _D, _D * f)
    wx = jnp.concatenate([W[:, :f]] * _D, axis=1)
    we = jnp.concatenate([W[:, f:]] * _D, axis=1)
    zb = jnp.zeros((1, _DD), jnp.float32)
    b2 = b.reshape(1, _DD).astype(jnp.float32)

    px = _project(x2, wx, zb, 1000)
    pe = _project(e2, we, b2, 1000)

    row = hyperedge_index[0]
    col = hyperedge_index[1]
    nnz = row.shape[0]

    sc_fn = _sc_build(nnz, px.size, pe.size)
    idx_flat, attr = sc_fn(px.reshape(-1), pe.reshape(-1), row, col)
    return idx_flat.reshape(2, _DD * nnz), attr
